# Initial kernel scaffold; baseline (speedup 1.0000x reference)
import jax
import jax.numpy as jnp
from jax import lax
from jax.experimental import pallas as pl
from jax.experimental.pallas import tpu as pltpu
from jax.experimental.pallas import tpu_sc as plsc

VOCAB = 100000
MAX_LEN = 200
EMBED_DIM = 64
BATCH = 1024

NC = 2
NS = 16
NW = NC * NS
SEQ_PER_W = BATCH // NW
CHUNK = 100
CHUNKS_PER_SEQ = MAX_LEN // CHUNK
LANES = 16
VPR = EMBED_DIM // LANES


def _body(x_hbm, tab_hbm, pos_hbm, out_hbm, idx_v, pos_v, rows_v, sem):
    wid = lax.axis_index("s") * NC + lax.axis_index("c")

    pltpu.sync_copy(x_hbm.at[wid], idx_v)
    pltpu.sync_copy(pos_hbm, pos_v)

    def seq_body(i, _):
        cp0 = pltpu.async_copy(
            tab_hbm.at[idx_v.at[2 * i]], rows_v.at[pl.ds(0, CHUNK)], sem)
        cp1 = pltpu.async_copy(
            tab_hbm.at[idx_v.at[2 * i + 1]], rows_v.at[pl.ds(CHUNK, CHUNK)], sem)
        cp0.wait()
        cp1.wait()

        def add_body(j, _):
            for c in range(VPR):
                sl = pl.ds(c * LANES, LANES)
                rows_v[j, sl] = rows_v[j, sl] + pos_v[j, sl]
            return 0

        lax.fori_loop(0, MAX_LEN, add_body, 0)

        base = (wid * SEQ_PER_W + i) * MAX_LEN
        pltpu.sync_copy(rows_v, out_hbm.at[pl.ds(base, MAX_LEN)])
        return 0

    lax.fori_loop(0, SEQ_PER_W, seq_body, 0)


@jax.jit
def kernel(x, token_table, pos_table):
    x_w = x.astype(jnp.int32).reshape(NW, SEQ_PER_W * CHUNKS_PER_SEQ, CHUNK)
    mesh = plsc.VectorSubcoreMesh(core_axis_name="c", subcore_axis_name="s")
    out = pl.kernel(
        _body,
        out_type=jax.ShapeDtypeStruct((BATCH * MAX_LEN, EMBED_DIM), jnp.float32),
        mesh=mesh,
        scratch_types=[
            pltpu.VMEM((SEQ_PER_W * CHUNKS_PER_SEQ, CHUNK), jnp.int32),
            pltpu.VMEM((MAX_LEN, EMBED_DIM), jnp.float32),
            pltpu.VMEM((MAX_LEN, EMBED_DIM), jnp.float32),
            pltpu.SemaphoreType.DMA,
        ],
    )(x_w, token_table, pos_table)
    return out.reshape(BATCH, MAX_LEN, EMBED_DIM)


# speedup vs baseline: 2.7198x; 2.7198x over previous
import jax
import jax.numpy as jnp
from jax import lax
from jax.experimental import pallas as pl
from jax.experimental.pallas import tpu as pltpu
from jax.experimental.pallas import tpu_sc as plsc

VOCAB = 100000
MAX_LEN = 200
EMBED_DIM = 64
BATCH = 1024

NC = 2
NS = 16
NW = NC * NS
SEQ_PER_W = BATCH // NW
CHUNK = 100
CHUNKS_PER_SEQ = MAX_LEN // CHUNK
LANES = 16
VPR = EMBED_DIM // LANES


def _body(x_hbm, tab_hbm, pos_hbm, out_hbm, idx_v, pos_v, rows_v, sem):
    wid = lax.axis_index("s") * NC + lax.axis_index("c")

    pltpu.sync_copy(x_hbm.at[wid], idx_v)
    pltpu.sync_copy(pos_hbm, pos_v)

    def seq_body(i, _):
        cp0 = pltpu.async_copy(
            tab_hbm.at[idx_v.at[2 * i]], rows_v.at[pl.ds(0, CHUNK)], sem)
        cp1 = pltpu.async_copy(
            tab_hbm.at[idx_v.at[2 * i + 1]], rows_v.at[pl.ds(CHUNK, CHUNK)], sem)
        cp0.wait()
        cp1.wait()

        def add_body(j, _):
            for c in range(VPR):
                sl = pl.ds(c * LANES, LANES)
                rows_v[j, sl] = rows_v[j, sl] + pos_v[j, sl]
            return 0

        lax.fori_loop(0, MAX_LEN, add_body, 0)

        base = (wid * SEQ_PER_W + i) * MAX_LEN
        pltpu.sync_copy(rows_v, out_hbm.at[pl.ds(base, MAX_LEN)])
        return 0

    lax.fori_loop(0, SEQ_PER_W, seq_body, 0)


@jax.jit
def kernel(x, token_table, pos_table):
    x_w = x.astype(jnp.int32).reshape(NW, SEQ_PER_W * CHUNKS_PER_SEQ, CHUNK)
    mesh = plsc.VectorSubcoreMesh(core_axis_name="c", subcore_axis_name="s")
    out = pl.kernel(
        _body,
        out_type=jax.ShapeDtypeStruct((BATCH * MAX_LEN, EMBED_DIM), jnp.float32),
        mesh=mesh,
        scratch_types=[
            pltpu.VMEM((SEQ_PER_W * CHUNKS_PER_SEQ, CHUNK), jnp.int32),
            pltpu.VMEM((MAX_LEN, EMBED_DIM), jnp.float32),
            pltpu.VMEM((MAX_LEN, EMBED_DIM), jnp.float32),
            pltpu.SemaphoreType.DMA,
        ],
        compiler_params=pltpu.CompilerParams(use_tc_tiling_on_sc=False),
    )(x_w, token_table, pos_table)
    return out.reshape(BATCH, MAX_LEN, EMBED_DIM)


# speedup vs baseline: 3.2107x; 1.1805x over previous
import jax
import jax.numpy as jnp
from jax import lax
from jax.experimental import pallas as pl
from jax.experimental.pallas import tpu as pltpu
from jax.experimental.pallas import tpu_sc as plsc

VOCAB = 100000
MAX_LEN = 200
EMBED_DIM = 64
BATCH = 1024

NC = 2
NS = 16
NW = NC * NS
SEQ_PER_W = BATCH // NW
CHUNK = 100
LANES = 16
VPR = EMBED_DIM // LANES
K = 4
NCHUNK = SEQ_PER_W // K
NHALF = NCHUNK // 2
STREAMS = K * MAX_LEN // CHUNK
IDX_ROWS = SEQ_PER_W * MAX_LEN // CHUNK


def _body(x_hbm, tab_hbm, pos_hbm, out_hbm, idx_v, pos_v, rows_v,
          gsem0, gsem1, ssem0, ssem1):
    wid = lax.axis_index("s") * NC + lax.axis_index("c")
    gsem = (gsem0, gsem1)
    ssem = (ssem0, ssem1)

    pltpu.sync_copy(x_hbm.at[wid], idx_v)
    pltpu.sync_copy(pos_hbm, pos_v)

    def issue_gather(c, b):
        for j in range(STREAMS):
            k, h = divmod(j, MAX_LEN // CHUNK)
            pltpu.async_copy(
                tab_hbm.at[idx_v.at[c * STREAMS + j]],
                rows_v.at[b, k, pl.ds(h * CHUNK, CHUNK)],
                gsem[b])

    def drain_gather(b):
        for _ in range(STREAMS):
            pltpu.make_async_copy(
                tab_hbm.at[idx_v.at[0]],
                rows_v.at[b, 0, pl.ds(0, CHUNK)],
                gsem[b]).wait()

    def issue_scatter(c, b):
        pltpu.async_copy(
            rows_v.at[b], out_hbm.at[pl.ds(wid * SEQ_PER_W + c * K, K)],
            ssem[b])

    def drain_scatter(b):
        pltpu.make_async_copy(
            rows_v.at[b], out_hbm.at[pl.ds(0, K)], ssem[b]).wait()

    def add_chunk(b):
        @plsc.parallel_loop(0, MAX_LEN, 1, unroll=4)
        def _(j):
            for c in range(VPR):
                sl = pl.ds(c * LANES, LANES)
                p = pos_v[j, sl]
                for k in range(K):
                    rows_v[b, k, j, sl] = rows_v[b, k, j, sl] + p

    issue_gather(0, 0)

    def step(i, _):
        c0 = 2 * i

        @pl.when(i > 0)
        def _():
            drain_scatter(1)

        issue_gather(c0 + 1, 1)

        drain_gather(0)
        add_chunk(0)
        issue_scatter(c0, 0)

        @pl.when(i < NHALF - 1)
        def _():
            drain_scatter(0)
            issue_gather(c0 + 2, 0)

        drain_gather(1)
        add_chunk(1)
        issue_scatter(c0 + 1, 1)
        return 0

    lax.fori_loop(0, NHALF, step, 0)
    drain_scatter(0)
    drain_scatter(1)


@jax.jit
def kernel(x, token_table, pos_table):
    x_w = x.astype(jnp.int32).reshape(NW, IDX_ROWS, CHUNK)
    mesh = plsc.VectorSubcoreMesh(core_axis_name="c", subcore_axis_name="s")
    return pl.kernel(
        _body,
        out_type=jax.ShapeDtypeStruct((BATCH, MAX_LEN, EMBED_DIM), jnp.float32),
        mesh=mesh,
        scratch_types=[
            pltpu.VMEM((IDX_ROWS, CHUNK), jnp.int32),
            pltpu.VMEM((MAX_LEN, EMBED_DIM), jnp.float32),
            pltpu.VMEM((2, K, MAX_LEN, EMBED_DIM), jnp.float32),
            pltpu.SemaphoreType.DMA,
            pltpu.SemaphoreType.DMA,
            pltpu.SemaphoreType.DMA,
            pltpu.SemaphoreType.DMA,
        ],
        compiler_params=pltpu.CompilerParams(use_tc_tiling_on_sc=False),
    )(x_w, token_table, pos_table)
